# Initial kernel scaffold; baseline (speedup 1.0000x reference)
import jax
import jax.numpy as jnp
from jax.experimental import pallas as pl


def kernel(x, edge_index, batch, W_head, b_head, W_blocks, b_blocks, W_fus, b_fus, W_p1, b_p1, W_p2, b_p2, W_p3, b_p3):
    raise NotImplementedError("write your pallas kernel here")


# speedup vs baseline: 1.4920x; 1.4920x over previous
import jax
import jax.numpy as jnp
from jax.experimental import pallas as pl

N = 10000
E = 320000
D = 128
N_BLOCKS = 7
N_CLASSES = 13
FUSION = D * N_BLOCKS


def _head_body(feats_ref, wfus_ref, bfus_ref, wp1a_ref, wp1b_ref, bp1_ref,
               wp2_ref, bp2_ref, wp3_ref, bp3_ref, out_ref):
    f = feats_ref[...]
    fus = jnp.max(jax.nn.relu(f @ wfus_ref[...] + bfus_ref[...]),
                  axis=1, keepdims=True)
    h = jax.nn.relu(f @ wp1a_ref[...] + fus * wp1b_ref[...] + bp1_ref[...])
    h = jax.nn.relu(h @ wp2_ref[...] + bp2_ref[...])
    out_ref[...] = h @ wp3_ref[...] + bp3_ref[...]


def _head(feats, W_fus, b_fus, W_p1, b_p1, W_p2, b_p2, W_p3, b_p3):
    ROWS = 1000
    grid = (N // ROWS,)
    out = pl.pallas_call(
        _head_body,
        grid=grid,
        in_specs=[
            pl.BlockSpec((ROWS, FUSION), lambda i: (i, 0)),
            pl.BlockSpec((FUSION, 1024), lambda i: (0, 0)),
            pl.BlockSpec((1, 1024), lambda i: (0, 0)),
            pl.BlockSpec((FUSION, 512), lambda i: (0, 0)),
            pl.BlockSpec((1, 512), lambda i: (0, 0)),
            pl.BlockSpec((1, 512), lambda i: (0, 0)),
            pl.BlockSpec((512, 256), lambda i: (0, 0)),
            pl.BlockSpec((1, 256), lambda i: (0, 0)),
            pl.BlockSpec((256, N_CLASSES), lambda i: (0, 0)),
            pl.BlockSpec((1, N_CLASSES), lambda i: (0, 0)),
        ],
        out_specs=pl.BlockSpec((ROWS, N_CLASSES), lambda i: (i, 0)),
        out_shape=jax.ShapeDtypeStruct((N, N_CLASSES), jnp.float32),
    )(feats, W_fus, b_fus[None, :], W_p1[:FUSION], W_p1[FUSION:],
      b_p1[None, :], W_p2, b_p2[None, :], W_p3, b_p3[None, :])
    return out


def kernel(x, edge_index, batch, W_head, b_head, W_blocks, b_blocks,
           W_fus, b_fus, W_p1, b_p1, W_p2, b_p2, W_p3, b_p3):
    src = edge_index[0]
    dst = edge_index[1]

    def mr_conv(h, W, b):
        segmax = jax.ops.segment_max(h[src], dst, num_segments=N)
        agg = jnp.where(segmax > -jnp.inf, segmax - h, 0.0)
        cat = jnp.concatenate([h, agg], axis=1)
        return jax.nn.relu(cat @ W + b)

    feats = [mr_conv(x, W_head, b_head)]
    for i in range(N_BLOCKS - 1):
        feats.append(mr_conv(feats[-1], W_blocks[i], b_blocks[i]) + feats[-1])
    feats = jnp.concatenate(feats, axis=1)
    return _head(feats, W_fus, b_fus, W_p1, b_p1, W_p2, b_p2, W_p3, b_p3)


# speedup vs baseline: 1.6810x; 1.1267x over previous
import functools

import jax
import jax.numpy as jnp
from jax import lax
from jax.experimental import pallas as pl
from jax.experimental.pallas import tpu as pltpu
from jax.experimental.pallas import tpu_sc as plsc

N = 10000
E = 320000
D = 128
NB = 7
NCLS = 13
FUS = D * NB

NWORK = 32
FPT = D // NWORK
CH = 6400
NCHUNK = E // CH
GPC = CH // 16
NEG = float("-inf")


def _segmax_body(xf, ef, out, xs, acc, sbuf, dbuf, det):
    c = lax.axis_index("c")
    s = lax.axis_index("s")
    w = s * 2 + c
    f0 = w * FPT

    pltpu.sync_copy(xf.at[pl.ds(f0 * N, FPT * N)], xs)

    neg = jnp.full((16,), NEG, jnp.float32)

    def init_body(i, _):
        acc[pl.ds(i * 16, 16)] = neg
        return 0

    lax.fori_loop(0, FPT * N // 16, init_body, 0)

    lane = lax.iota(jnp.int32, 16)
    offs = [jnp.full((16,), f * N, jnp.int32) for f in range(FPT)]

    for ci in range(NCHUNK):
        pltpu.sync_copy(ef.at[pl.ds(ci * CH, CH)], sbuf)
        pltpu.sync_copy(ef.at[pl.ds(E + ci * CH, CH)], dbuf)

        def group(g, _):
            src16 = sbuf[pl.ds(g * 16, 16)]
            dst16 = dbuf[pl.ds(g * 16, 16)]
            xg = [plsc.load_gather(xs, [src16 + offs[f]]) for f in range(FPT)]
            plsc.store_scatter(det, [dst16], lane)
            rb = plsc.load_gather(det, [dst16])
            nodup = jnp.max(jnp.where(rb == lane, 0, 1)) == 0
            didx = [dst16 + offs[f] for f in range(FPT)]

            def fast():
                for f in range(FPT):
                    a = plsc.load_gather(acc, [didx[f]])
                    plsc.store_scatter(acc, [didx[f]], jnp.maximum(a, xg[f]))

            def slow():
                def ebody(e, _):
                    m = lane == e
                    for f in range(FPT):
                        a = plsc.load_gather(acc, [didx[f]])
                        plsc.store_scatter(acc, [didx[f]],
                                           jnp.maximum(a, xg[f]), mask=m)
                    return 0
                lax.fori_loop(0, 16, ebody, 0)

            lax.cond(nodup, fast, slow)
            return 0

        lax.fori_loop(0, GPC, group, 0)

    pltpu.sync_copy(acc, out.at[pl.ds(f0 * N, FPT * N)])


@functools.partial(
    pl.kernel,
    out_type=jax.ShapeDtypeStruct((D * N,), jnp.float32),
    mesh=plsc.VectorSubcoreMesh(core_axis_name="c", subcore_axis_name="s"),
    compiler_params=pltpu.CompilerParams(needs_layout_passes=False),
    scratch_types=[
        pltpu.VMEM((FPT * N,), jnp.float32),
        pltpu.VMEM((FPT * N,), jnp.float32),
        pltpu.VMEM((CH,), jnp.int32),
        pltpu.VMEM((CH,), jnp.int32),
        pltpu.VMEM((N,), jnp.int32),
    ],
)
def _segmax_flat(xf, ef, out, xs, acc, sbuf, dbuf, det):
    _segmax_body(xf, ef, out, xs, acc, sbuf, dbuf, det)


def _segmax(xT, eflat):
    return _segmax_flat(xT.reshape(-1), eflat).reshape(D, N)


CT = 2048


def _block_body(res, xT_ref, sg_ref, wt_ref, b_ref, o_ref):
    x = xT_ref[...]
    sg = sg_ref[...]
    agg = jnp.where(sg > NEG, sg - x, 0.0)
    cat = jnp.concatenate([x, agg], axis=0)
    y = jnp.dot(wt_ref[...], cat, preferred_element_type=jnp.float32)
    y = jnp.maximum(y + b_ref[...], 0.0)
    if res:
        y = y + x
    o_ref[...] = y


def _block(xT, sg, Wt, b2, res):
    return pl.pallas_call(
        functools.partial(_block_body, res),
        grid=(pl.cdiv(N, CT),),
        in_specs=[
            pl.BlockSpec((D, CT), lambda i: (0, i)),
            pl.BlockSpec((D, CT), lambda i: (0, i)),
            pl.BlockSpec((D, 2 * D), lambda i: (0, 0)),
            pl.BlockSpec((D, 1), lambda i: (0, 0)),
        ],
        out_specs=pl.BlockSpec((D, CT), lambda i: (0, i)),
        out_shape=jax.ShapeDtypeStruct((D, N), jnp.float32),
    )(xT, sg, Wt, b2)


CTH = 1024


def _head_body(f1, f2, f3, f4, f5, f6, f7, wfus, bfus, wp1a, wp1b, bp1,
               wp2, bp2, wp3, bp3, o_ref):
    feats = jnp.concatenate(
        [f1[...], f2[...], f3[...], f4[...], f5[...], f6[...], f7[...]],
        axis=0)
    t = jnp.dot(wfus[...], feats, preferred_element_type=jnp.float32)
    fus = jnp.max(jnp.maximum(t + bfus[...], 0.0), axis=0, keepdims=True)
    h = jnp.dot(wp1a[...], feats, preferred_element_type=jnp.float32)
    h = jnp.maximum(h + wp1b[...] * fus + bp1[...], 0.0)
    h = jnp.dot(wp2[...], h, preferred_element_type=jnp.float32)
    h = jnp.maximum(h + bp2[...], 0.0)
    o_ref[...] = jnp.dot(wp3[...], h,
                         preferred_element_type=jnp.float32) + bp3[...]


def _head(ys, W_fus, b_fus, W_p1, b_p1, W_p2, b_p2, W_p3, b_p3):
    col = lambda i: (0, i)
    fix = lambda i: (0, 0)
    return pl.pallas_call(
        _head_body,
        grid=(pl.cdiv(N, CTH),),
        in_specs=[pl.BlockSpec((D, CTH), col)] * NB + [
            pl.BlockSpec((1024, FUS), fix),
            pl.BlockSpec((1024, 1), fix),
            pl.BlockSpec((512, FUS), fix),
            pl.BlockSpec((512, 1), fix),
            pl.BlockSpec((512, 1), fix),
            pl.BlockSpec((256, 512), fix),
            pl.BlockSpec((256, 1), fix),
            pl.BlockSpec((NCLS, 256), fix),
            pl.BlockSpec((NCLS, 1), fix),
        ],
        out_specs=pl.BlockSpec((NCLS, CTH), col),
        out_shape=jax.ShapeDtypeStruct((NCLS, N), jnp.float32),
    )(*ys, W_fus.T, b_fus[:, None], W_p1[:FUS].T, W_p1[FUS:].T,
      b_p1[:, None], W_p2.T, b_p2[:, None], W_p3.T, b_p3[:, None])


def kernel(x, edge_index, batch, W_head, b_head, W_blocks, b_blocks,
           W_fus, b_fus, W_p1, b_p1, W_p2, b_p2, W_p3, b_p3):
    xT = x.T
    eflat = edge_index.reshape(-1)
    sg = _segmax(xT, eflat)
    ys = [_block(xT, sg, W_head.T, b_head[:, None], res=False)]
    for i in range(NB - 1):
        sg = _segmax(ys[-1], eflat)
        ys.append(_block(ys[-1], sg, W_blocks[i].T, b_blocks[i][:, None],
                         res=True))
    outT = _head(ys, W_fus, b_fus, W_p1, b_p1, W_p2, b_p2, W_p3, b_p3)
    return outT.T


# speedup vs baseline: 2.6462x; 1.5741x over previous
import functools

import jax
import jax.numpy as jnp
from jax import lax
from jax.experimental import pallas as pl
from jax.experimental.pallas import tpu as pltpu
from jax.experimental.pallas import tpu_sc as plsc

N = 10000
E = 320000
D = 128
NB = 7
NCLS = 13
FUS = D * NB

NWORK = 32
FPT = D // NWORK
CH = 6400
NCHUNK = E // CH
GPC = CH // 16
NEG = float("-inf")


DETP = N + 16
UNROLL = 2


def _segmax_body(xf, ef, out, xs, acc, sbuf, dbuf, lbs, lbd, det):
    c = lax.axis_index("c")
    s = lax.axis_index("s")
    w = s * 2 + c
    f0 = w * FPT

    pltpu.sync_copy(xf.at[pl.ds(f0 * N, FPT * N)], xs)

    neg = jnp.full((16,), NEG, jnp.float32)

    def init_body(i, _):
        acc[pl.ds(i * 16, 16)] = neg
        return 0

    lax.fori_loop(0, FPT * N // 16, init_body, 0)

    lane = lax.iota(jnp.int32, 16)
    offsN = [jnp.full((16,), f * N, jnp.int32) for f in range(FPT)]

    def update(srcv, dstv, accv, det_base, off):
        xg = [plsc.load_gather(xs, [srcv + offsN[f]]) for f in range(FPT)]
        dix = dstv + jnp.full((16,), det_base, jnp.int32)
        plsc.store_scatter(det, [dix], lane)
        rb = plsc.load_gather(det, [dix])
        win = rb == lane
        wm = win & (dstv < N)
        lm = (~win) & (dstv < N)
        for f in range(FPT):
            aix = accv + offsN[f]
            a = plsc.load_gather(acc, [aix])
            plsc.store_scatter(acc, [aix], jnp.maximum(a, xg[f]), mask=wm)
        cnt = plsc.all_reduce_population_count(lm)[0]
        plsc.store_compressed(lbs.at[pl.ds(off, 16)], srcv, mask=lm)
        plsc.store_compressed(lbd.at[pl.ds(off, 16)], dstv, mask=lm)
        return off + cnt

    for ci in range(NCHUNK):
        pltpu.sync_copy(ef.at[pl.ds(ci * CH, CH)], sbuf)
        pltpu.sync_copy(ef.at[pl.ds(E + ci * CH, CH)], dbuf)

        def group(g, off):
            for u in range(UNROLL):
                b = g * (16 * UNROLL) + u * 16
                srcv = sbuf[pl.ds(b, 16)]
                dstv = dbuf[pl.ds(b, 16)]
                off = update(srcv, dstv, dstv, u * DETP, off)
            return off

        nlost = lax.fori_loop(0, GPC // UNROLL, group, 0)

        def drain_cond(L):
            return L > 0

        def drain_pass(L):
            def gbody(k, off):
                base = k * 16
                srcv = lbs[pl.ds(base, 16)]
                dstv = lbd[pl.ds(base, 16)]
                valid = lane < (L - base)
                dstv = jnp.where(valid, dstv, N + lane)
                srcv = jnp.where(valid, srcv, 0)
                accv = jnp.where(valid, dstv, 0)
                return update(srcv, dstv, accv, 0, off)

            ng = (L + 15) // 16
            return lax.fori_loop(0, ng, gbody, 0)

        lax.while_loop(drain_cond, drain_pass, nlost)

    pltpu.sync_copy(acc, out.at[pl.ds(f0 * N, FPT * N)])


@functools.partial(
    pl.kernel,
    out_type=jax.ShapeDtypeStruct((D * N,), jnp.float32),
    mesh=plsc.VectorSubcoreMesh(core_axis_name="c", subcore_axis_name="s"),
    compiler_params=pltpu.CompilerParams(needs_layout_passes=False),
    scratch_types=[
        pltpu.VMEM((FPT * N,), jnp.float32),
        pltpu.VMEM((FPT * N,), jnp.float32),
        pltpu.VMEM((CH,), jnp.int32),
        pltpu.VMEM((CH,), jnp.int32),
        pltpu.VMEM((CH + 16,), jnp.int32),
        pltpu.VMEM((CH + 16,), jnp.int32),
        pltpu.VMEM((UNROLL * DETP,), jnp.int32),
    ],
)
def _segmax_flat(xf, ef, out, xs, acc, sbuf, dbuf, lbs, lbd, det):
    _segmax_body(xf, ef, out, xs, acc, sbuf, dbuf, lbs, lbd, det)


def _segmax(xT, eflat):
    return _segmax_flat(xT.reshape(-1), eflat).reshape(D, N)


CT = 2048


def _block_body(res, xT_ref, sg_ref, wt_ref, b_ref, o_ref):
    x = xT_ref[...]
    sg = sg_ref[...]
    agg = jnp.where(sg > NEG, sg - x, 0.0)
    cat = jnp.concatenate([x, agg], axis=0)
    y = jnp.dot(wt_ref[...], cat, preferred_element_type=jnp.float32)
    y = jnp.maximum(y + b_ref[...], 0.0)
    if res:
        y = y + x
    o_ref[...] = y


def _block(xT, sg, Wt, b2, res):
    return pl.pallas_call(
        functools.partial(_block_body, res),
        grid=(pl.cdiv(N, CT),),
        in_specs=[
            pl.BlockSpec((D, CT), lambda i: (0, i)),
            pl.BlockSpec((D, CT), lambda i: (0, i)),
            pl.BlockSpec((D, 2 * D), lambda i: (0, 0)),
            pl.BlockSpec((D, 1), lambda i: (0, 0)),
        ],
        out_specs=pl.BlockSpec((D, CT), lambda i: (0, i)),
        out_shape=jax.ShapeDtypeStruct((D, N), jnp.float32),
    )(xT, sg, Wt, b2)


CTH = 1024


def _head_body(f1, f2, f3, f4, f5, f6, f7, wfus, bfus, wp1a, wp1b, bp1,
               wp2, bp2, wp3, bp3, o_ref):
    feats = jnp.concatenate(
        [f1[...], f2[...], f3[...], f4[...], f5[...], f6[...], f7[...]],
        axis=0)
    t = jnp.dot(wfus[...], feats, preferred_element_type=jnp.float32)
    fus = jnp.max(jnp.maximum(t + bfus[...], 0.0), axis=0, keepdims=True)
    h = jnp.dot(wp1a[...], feats, preferred_element_type=jnp.float32)
    h = jnp.maximum(h + wp1b[...] * fus + bp1[...], 0.0)
    h = jnp.dot(wp2[...], h, preferred_element_type=jnp.float32)
    h = jnp.maximum(h + bp2[...], 0.0)
    o_ref[...] = jnp.dot(wp3[...], h,
                         preferred_element_type=jnp.float32) + bp3[...]


def _head(ys, W_fus, b_fus, W_p1, b_p1, W_p2, b_p2, W_p3, b_p3):
    col = lambda i: (0, i)
    fix = lambda i: (0, 0)
    return pl.pallas_call(
        _head_body,
        grid=(pl.cdiv(N, CTH),),
        in_specs=[pl.BlockSpec((D, CTH), col)] * NB + [
            pl.BlockSpec((1024, FUS), fix),
            pl.BlockSpec((1024, 1), fix),
            pl.BlockSpec((512, FUS), fix),
            pl.BlockSpec((512, 1), fix),
            pl.BlockSpec((512, 1), fix),
            pl.BlockSpec((256, 512), fix),
            pl.BlockSpec((256, 1), fix),
            pl.BlockSpec((NCLS, 256), fix),
            pl.BlockSpec((NCLS, 1), fix),
        ],
        out_specs=pl.BlockSpec((NCLS, CTH), col),
        out_shape=jax.ShapeDtypeStruct((NCLS, N), jnp.float32),
    )(*ys, W_fus.T, b_fus[:, None], W_p1[:FUS].T, W_p1[FUS:].T,
      b_p1[:, None], W_p2.T, b_p2[:, None], W_p3.T, b_p3[:, None])


def kernel(x, edge_index, batch, W_head, b_head, W_blocks, b_blocks,
           W_fus, b_fus, W_p1, b_p1, W_p2, b_p2, W_p3, b_p3):
    xT = x.T
    eflat = edge_index.reshape(-1)
    sg = _segmax(xT, eflat)
    ys = [_block(xT, sg, W_head.T, b_head[:, None], res=False)]
    for i in range(NB - 1):
        sg = _segmax(ys[-1], eflat)
        ys.append(_block(ys[-1], sg, W_blocks[i].T, b_blocks[i][:, None],
                         res=True))
    outT = _head(ys, W_fus, b_fus, W_p1, b_p1, W_p2, b_p2, W_p3, b_p3)
    return outT.T
